# Initial kernel scaffold; baseline (speedup 1.0000x reference)
import jax
import jax.numpy as jnp
from jax.experimental import pallas as pl


def kernel(x, x_emb, x_cal_emb, tables, cal_table):
    raise NotImplementedError("write your pallas kernel here")


# speedup vs baseline: 1.0724x; 1.0724x over previous
import functools

import jax
import jax.numpy as jnp
from jax import lax
from jax.experimental import pallas as pl
from jax.experimental.pallas import tpu as pltpu
from jax.experimental.pallas import tpu_sc as plsc

_NF = 26
_V = 100000
_ED = 32
_CV = 366
_CD = 16
_B = 1024
_T = 50
_IN = 16
_ROWS = _T * _B
_TILE = 128
_NT = _ROWS // _TILE
_TPB = _B // _TILE
_OUT_D = _IN + _NF * _ED + 2 * _CD

_info = plsc.get_sparse_core_info()
_NC = _info.num_cores
_NS = _info.num_subcores
_NW = _NC * _NS


def _sc_body(x_hbm, xe_hbm, xc_hbm, tab_hbm, cal_hbm, out_hbm,
             idx_v, gat_v, x_v, cal_v, sem):
    wid = lax.axis_index("s") * _NC + lax.axis_index("c")
    ntiles = (_NT - wid + _NW - 1) // _NW

    def tile_body(i, carry):
        tile = wid + i * _NW
        r0 = tile * _TILE
        t = tile // _TPB
        b0 = (tile % _TPB) * _TILE
        pltpu.sync_copy(x_hbm.at[pl.ds(b0, _TILE), t, :], x_v)
        pltpu.sync_copy(x_v, out_hbm.at[pl.ds(r0, _TILE), pl.ds(0, _IN)])

        def field_body(f, c2):
            pltpu.sync_copy(xe_hbm.at[f, pl.ds(r0, _TILE)], idx_v)
            pltpu.async_copy(tab_hbm.at[idx_v], gat_v, sem).wait()
            col = _IN + f * _ED
            pltpu.sync_copy(gat_v, out_hbm.at[pl.ds(r0, _TILE), pl.ds(col, _ED)])
            return c2

        lax.fori_loop(0, _NF, field_body, 0)

        def cal_body(c, c2):
            pltpu.sync_copy(xc_hbm.at[c, pl.ds(r0, _TILE)], idx_v)
            pltpu.async_copy(cal_hbm.at[idx_v], cal_v, sem).wait()
            col = _IN + _NF * _ED + c * _CD
            pltpu.sync_copy(cal_v, out_hbm.at[pl.ds(r0, _TILE), pl.ds(col, _CD)])
            return c2

        lax.fori_loop(0, 2, cal_body, 0)
        return carry

    lax.fori_loop(0, ntiles, tile_body, 0)


_mesh = plsc.VectorSubcoreMesh(core_axis_name="c", subcore_axis_name="s")

_sc_call = pl.kernel(
    _sc_body,
    out_type=jax.ShapeDtypeStruct((_ROWS, _OUT_D), jnp.float32),
    mesh=_mesh,
    compiler_params=pltpu.CompilerParams(use_tc_tiling_on_sc=False),
    scratch_types=[
        pltpu.VMEM((_TILE,), jnp.int32),
        pltpu.VMEM((_TILE, _ED), jnp.float32),
        pltpu.VMEM((_TILE, _IN), jnp.float32),
        pltpu.VMEM((_TILE, _CD), jnp.float32),
        pltpu.SemaphoreType.DMA,
    ],
)


def kernel(x, x_emb, x_cal_emb, tables, cal_table):
    xe = jnp.transpose(x_emb.astype(jnp.int32), (2, 1, 0)).reshape(_NF, _ROWS)
    xe = xe + (jnp.arange(_NF, dtype=jnp.int32) * _V)[:, None]
    xc = jnp.transpose(x_cal_emb.astype(jnp.int32), (2, 1, 0)).reshape(2, _ROWS)
    tab = tables.reshape(_NF * _V, _ED)
    out = _sc_call(x, xe, xc, tab, cal_table)
    return out.reshape(_T, _B, _OUT_D)


# speedup vs baseline: 1.3345x; 1.2445x over previous
import functools

import jax
import jax.numpy as jnp
from jax import lax
from jax.experimental import pallas as pl
from jax.experimental.pallas import tpu as pltpu
from jax.experimental.pallas import tpu_sc as plsc

_NF = 26
_V = 100000
_ED = 32
_CV = 366
_CD = 16
_B = 1024
_T = 50
_IN = 16
_ROWS = _T * _B
_TILE = 128
_NT = _ROWS // _TILE
_TPB = _B // _TILE
_OUT_D = _IN + _NF * _ED + 2 * _CD

_info = plsc.get_sparse_core_info()
_NC = _info.num_cores
_NS = _info.num_subcores
_NW = _NC * _NS


def _sc_body(x_hbm, xe_hbm, xc_hbm, tab_hbm, cal_hbm, out_hbm,
             idx_v, cidx_v, gat_v, x_v, cal_v, sem, wsem):
    wid = lax.axis_index("s") * _NC + lax.axis_index("c")
    ntiles = (_NT - wid + _NW - 1) // _NW

    def tile_body(i, carry):
        tile = wid + i * _NW
        r0 = tile * _TILE
        t = tile // _TPB
        b0 = (tile % _TPB) * _TILE
        pltpu.sync_copy(xe_hbm.at[:, pl.ds(r0, _TILE)], idx_v)
        pltpu.sync_copy(xc_hbm.at[:, pl.ds(r0, _TILE)], cidx_v)
        gathers = [pltpu.async_copy(tab_hbm.at[idx_v.at[f]], gat_v.at[f], sem)
                   for f in range(_NF)]
        g_cal = [pltpu.async_copy(cal_hbm.at[cidx_v.at[c]], cal_v.at[c], sem)
                 for c in range(2)]
        pltpu.sync_copy(x_hbm.at[pl.ds(b0, _TILE), t, :], x_v)
        w_x = pltpu.async_copy(x_v, out_hbm.at[pl.ds(r0, _TILE), pl.ds(0, _IN)],
                               wsem)
        writes = []
        for f in range(_NF):
            gathers[f].wait()
            writes.append(pltpu.async_copy(
                gat_v.at[f], out_hbm.at[pl.ds(r0, _TILE),
                                        pl.ds(_IN + f * _ED, _ED)], wsem))
        for c in range(2):
            g_cal[c].wait()
            writes.append(pltpu.async_copy(
                cal_v.at[c], out_hbm.at[pl.ds(r0, _TILE),
                                        pl.ds(_IN + _NF * _ED + c * _CD, _CD)],
                wsem))
        w_x.wait()
        for w in writes:
            w.wait()
        return carry

    lax.fori_loop(0, ntiles, tile_body, 0)


_mesh = plsc.VectorSubcoreMesh(core_axis_name="c", subcore_axis_name="s")

_sc_call = pl.kernel(
    _sc_body,
    out_type=jax.ShapeDtypeStruct((_ROWS, _OUT_D), jnp.float32),
    mesh=_mesh,
    compiler_params=pltpu.CompilerParams(use_tc_tiling_on_sc=False),
    scratch_types=[
        pltpu.VMEM((_NF, _TILE), jnp.int32),
        pltpu.VMEM((2, _TILE), jnp.int32),
        pltpu.VMEM((_NF, _TILE, _ED), jnp.float32),
        pltpu.VMEM((_TILE, _IN), jnp.float32),
        pltpu.VMEM((2, _TILE, _CD), jnp.float32),
        pltpu.SemaphoreType.DMA,
        pltpu.SemaphoreType.DMA,
    ],
)


def kernel(x, x_emb, x_cal_emb, tables, cal_table):
    xe = jnp.transpose(x_emb.astype(jnp.int32), (2, 1, 0)).reshape(_NF, _ROWS)
    xe = xe + (jnp.arange(_NF, dtype=jnp.int32) * _V)[:, None]
    xc = jnp.transpose(x_cal_emb.astype(jnp.int32), (2, 1, 0)).reshape(2, _ROWS)
    tab = tables.reshape(_NF * _V, _ED)
    out = _sc_call(x, xe, xc, tab, cal_table)
    return out.reshape(_T, _B, _OUT_D)


# speedup vs baseline: 1.3399x; 1.0040x over previous
import jax
import jax.numpy as jnp
from jax import lax
from jax.experimental import pallas as pl
from jax.experimental.pallas import tpu as pltpu
from jax.experimental.pallas import tpu_sc as plsc

_NF = 26
_V = 100000
_ED = 32
_CV = 366
_CD = 16
_B = 1024
_T = 50
_IN = 16
_ROWS = _T * _B
_TILE = 64
_NT = _ROWS // _TILE
_TPB = _B // _TILE
_OUT_D = _IN + _NF * _ED + 2 * _CD

_info = plsc.get_sparse_core_info()
_NC = _info.num_cores
_NS = _info.num_subcores
_NW = _NC * _NS
_NTW = _NT // _NW


def _sc_body(x_hbm, xe_hbm, xc_hbm, tab_hbm, cal_hbm, out_hbm,
             idx_v, cidx_v, gat_v, x_v, cal_v, gsem, wsem0, wsem1):
    wid = lax.axis_index("s") * _NC + lax.axis_index("c")

    def fire(k, b, wsem):
        tile = wid + k * _NW
        r0 = tile * _TILE
        t = tile // _TPB
        b0 = (tile % _TPB) * _TILE
        pltpu.sync_copy(xe_hbm.at[:, pl.ds(r0, _TILE)], idx_v.at[b])
        pltpu.sync_copy(xc_hbm.at[:, pl.ds(r0, _TILE)], cidx_v.at[b])
        gathers = [pltpu.async_copy(tab_hbm.at[idx_v.at[b, f]],
                                    gat_v.at[b, f], gsem)
                   for f in range(_NF)]
        gcal = [pltpu.async_copy(cal_hbm.at[cidx_v.at[b, c]],
                                 cal_v.at[b, c], gsem)
                for c in range(2)]
        pltpu.sync_copy(x_hbm.at[pl.ds(b0, _TILE), t, :], x_v.at[b])
        pltpu.async_copy(x_v.at[b],
                         out_hbm.at[pl.ds(r0, _TILE), pl.ds(0, _IN)], wsem)
        for f in range(_NF):
            gathers[f].wait()
            pltpu.async_copy(
                gat_v.at[b, f],
                out_hbm.at[pl.ds(r0, _TILE), pl.ds(_IN + f * _ED, _ED)], wsem)
        for c in range(2):
            gcal[c].wait()
            pltpu.async_copy(
                cal_v.at[b, c],
                out_hbm.at[pl.ds(r0, _TILE),
                           pl.ds(_IN + _NF * _ED + c * _CD, _CD)], wsem)

    def drain(b, wsem):
        pltpu.make_async_copy(out_hbm.at[pl.ds(0, _TILE), pl.ds(0, _IN)],
                              x_v.at[b], wsem).wait()
        for f in range(_NF):
            pltpu.make_async_copy(
                out_hbm.at[pl.ds(0, _TILE), pl.ds(_IN + f * _ED, _ED)],
                gat_v.at[b, f], wsem).wait()
        for c in range(2):
            pltpu.make_async_copy(
                out_hbm.at[pl.ds(0, _TILE), pl.ds(0, _CD)],
                cal_v.at[b, c], wsem).wait()

    fire(0, 0, wsem0)
    fire(1, 1, wsem1)

    def body(i, carry):
        k0 = 2 * i + 2
        drain(0, wsem0)
        fire(k0, 0, wsem0)
        drain(1, wsem1)
        fire(k0 + 1, 1, wsem1)
        return carry

    lax.fori_loop(0, (_NTW - 3) // 2, body, 0)
    drain(0, wsem0)
    fire(_NTW - 1, 0, wsem0)
    drain(1, wsem1)
    drain(0, wsem0)


_mesh = plsc.VectorSubcoreMesh(core_axis_name="c", subcore_axis_name="s")

_sc_call = pl.kernel(
    _sc_body,
    out_type=jax.ShapeDtypeStruct((_ROWS, _OUT_D), jnp.float32),
    mesh=_mesh,
    compiler_params=pltpu.CompilerParams(use_tc_tiling_on_sc=False),
    scratch_types=[
        pltpu.VMEM((2, _NF, _TILE), jnp.int32),
        pltpu.VMEM((2, 2, _TILE), jnp.int32),
        pltpu.VMEM((2, _NF, _TILE, _ED), jnp.float32),
        pltpu.VMEM((2, _TILE, _IN), jnp.float32),
        pltpu.VMEM((2, 2, _TILE, _CD), jnp.float32),
        pltpu.SemaphoreType.DMA,
        pltpu.SemaphoreType.DMA,
        pltpu.SemaphoreType.DMA,
    ],
)


def kernel(x, x_emb, x_cal_emb, tables, cal_table):
    xe = jnp.transpose(x_emb.astype(jnp.int32), (2, 1, 0)).reshape(_NF, _ROWS)
    xe = xe + (jnp.arange(_NF, dtype=jnp.int32) * _V)[:, None]
    xc = jnp.transpose(x_cal_emb.astype(jnp.int32), (2, 1, 0)).reshape(2, _ROWS)
    tab = tables.reshape(_NF * _V, _ED)
    out = _sc_call(x, xe, xc, tab, cal_table)
    return out.reshape(_T, _B, _OUT_D)


# speedup vs baseline: 1.3490x; 1.0068x over previous
import jax
import jax.numpy as jnp
from jax import lax
from jax.experimental import pallas as pl
from jax.experimental.pallas import tpu as pltpu
from jax.experimental.pallas import tpu_sc as plsc

_NF = 26
_V = 100000
_CV = 366
_ED = 32
_CD = 16
_B = 1024
_T = 50
_IN = 16
_ROWS = _T * _B
_TILE = 64
_NT = _ROWS // _TILE
_TPB = _B // _TILE
_EMBW = _NF * _ED
_OUT_D = _IN + _EMBW + 2 * _CD
_GN = _TILE * _NF
_CN = _TILE * 2

_info = plsc.get_sparse_core_info()
_NC = _info.num_cores
_NS = _info.num_subcores
_NW = _NC * _NS
_NTW = _NT // _NW


def _sc_body(x_hbm, xe_hbm, xc_hbm, tab_hbm, cal_hbm, out_hbm,
             idx_v, cidx_v, gat0, gat1, cgat0, cgat1, x_v,
             gsem, wsem0, wsem1):
    wid = lax.axis_index("s") * _NC + lax.axis_index("c")
    gats = (gat0, gat1)
    cgats = (cgat0, cgat1)

    def fire(k, b, wsem):
        tile = wid + k * _NW
        r0 = tile * _TILE
        t = tile // _TPB
        b0 = (tile % _TPB) * _TILE
        pltpu.sync_copy(xe_hbm.at[tile], idx_v.at[b])
        pltpu.sync_copy(xc_hbm.at[tile], cidx_v.at[b])
        g_emb = pltpu.async_copy(tab_hbm.at[idx_v.at[b]], gats[b], gsem)
        g_cal = pltpu.async_copy(cal_hbm.at[cidx_v.at[b]], cgats[b], gsem)
        pltpu.sync_copy(x_hbm.at[pl.ds(b0, _TILE), t, :], x_v.at[b])
        pltpu.async_copy(x_v.at[b],
                         out_hbm.at[pl.ds(r0, _TILE), pl.ds(0, _IN)], wsem)
        g_emb.wait()
        for f in range(_NF):
            pltpu.async_copy(
                gats[b].at[pl.ds(f * _TILE, _TILE)],
                out_hbm.at[pl.ds(r0, _TILE), pl.ds(_IN + f * _ED, _ED)], wsem)
        g_cal.wait()
        for c in range(2):
            pltpu.async_copy(
                cgats[b].at[pl.ds(c * _TILE, _TILE)],
                out_hbm.at[pl.ds(r0, _TILE),
                           pl.ds(_IN + _EMBW + c * _CD, _CD)], wsem)

    def drain(b, wsem):
        pltpu.make_async_copy(out_hbm.at[pl.ds(0, _TILE), pl.ds(0, _IN)],
                              x_v.at[b], wsem).wait()
        for f in range(_NF):
            pltpu.make_async_copy(
                out_hbm.at[pl.ds(0, _TILE), pl.ds(_IN + f * _ED, _ED)],
                gats[b].at[pl.ds(f * _TILE, _TILE)], wsem).wait()
        for c in range(2):
            pltpu.make_async_copy(
                out_hbm.at[pl.ds(0, _TILE), pl.ds(_IN + _EMBW + c * _CD, _CD)],
                cgats[b].at[pl.ds(c * _TILE, _TILE)], wsem).wait()

    fire(0, 0, wsem0)
    fire(1, 1, wsem1)

    def body(i, carry):
        k0 = 2 * i + 2
        drain(0, wsem0)
        fire(k0, 0, wsem0)
        drain(1, wsem1)
        fire(k0 + 1, 1, wsem1)
        return carry

    lax.fori_loop(0, (_NTW - 3) // 2, body, 0)
    drain(0, wsem0)
    fire(_NTW - 1, 0, wsem0)
    drain(1, wsem1)
    drain(0, wsem0)


_mesh = plsc.VectorSubcoreMesh(core_axis_name="c", subcore_axis_name="s")

_sc_call = pl.kernel(
    _sc_body,
    out_type=jax.ShapeDtypeStruct((_ROWS, _OUT_D), jnp.float32),
    mesh=_mesh,
    compiler_params=pltpu.CompilerParams(use_tc_tiling_on_sc=False),
    scratch_types=[
        pltpu.VMEM((2, _GN), jnp.int32),
        pltpu.VMEM((2, _CN), jnp.int32),
        pltpu.VMEM((_GN, _ED), jnp.float32),
        pltpu.VMEM((_GN, _ED), jnp.float32),
        pltpu.VMEM((_CN, _CD), jnp.float32),
        pltpu.VMEM((_CN, _CD), jnp.float32),
        pltpu.VMEM((2, _TILE, _IN), jnp.float32),
        pltpu.SemaphoreType.DMA,
        pltpu.SemaphoreType.DMA,
        pltpu.SemaphoreType.DMA,
    ],
)


def kernel(x, x_emb, x_cal_emb, tables, cal_table):
    xe = jnp.transpose(x_emb.astype(jnp.int32), (1, 0, 2))
    xe = xe + (jnp.arange(_NF, dtype=jnp.int32) * _V)
    xe = xe.reshape(_NT, _TILE, _NF)
    xe = jnp.transpose(xe, (0, 2, 1)).reshape(_NT, _GN)
    xc = jnp.transpose(x_cal_emb.astype(jnp.int32), (1, 0, 2))
    xc = xc.reshape(_NT, _TILE, 2)
    xc = jnp.transpose(xc, (0, 2, 1)).reshape(_NT, _CN)
    tab = tables.reshape(_NF * _V, _ED)
    out = _sc_call(x, xe, xc, tab, cal_table)
    return out.reshape(_T, _B, _OUT_D)
